# baseline (speedup 1.0000x reference)
import functools

import jax
import jax.numpy as jnp
from jax import lax
from jax.experimental import pallas as pl
from jax.experimental.pallas import tpu as pltpu
from jax.experimental.pallas import tpu_sc as plsc

_VOCAB = 1000
_VP = 1024
_NLC = _VP // 128
_TAIL = _VOCAB - 7 * 128
_C = 64
_CP = 128
_T = 8
_B = 4096

_NC = 2
_NS = 16
_NW = _NC * _NS
_BPW = _B // _NW
_CB = 4
_CR = _CB * _T
_NCHUNK = _BPW // _CB
_NKEY = _T * _VOCAB


def _tt_body(tok_ref, pos_ref, w_ref, b_ref, o_ref):
    t = pl.program_id(0)
    w = w_ref[...]
    p = jnp.dot(pos_ref[pl.ds(t, 1), :], w, preferred_element_type=jnp.float32)
    o_ref[...] = (jnp.dot(tok_ref[...], w, preferred_element_type=jnp.float32)
                  + p + b_ref[...])


def _sc_emit(table, keys3):
    mesh = plsc.VectorSubcoreMesh(core_axis_name="c", subcore_axis_name="s")

    @functools.partial(
        pl.kernel,
        mesh=mesh,
        out_type=jax.ShapeDtypeStruct((_B, _T, _VOCAB), jnp.float32),
        scratch_types=[
            pltpu.VMEM((_NCHUNK * _NLC, _CR), jnp.int32),
            pltpu.VMEM((_NLC, _CR, 128), jnp.float32),
            pltpu.VMEM((_NLC, _CR, 128), jnp.float32),
            pltpu.VMEM((_CR, _TAIL), jnp.float32),
            pltpu.VMEM((_CR, _TAIL), jnp.float32),
            pltpu.SemaphoreType.DMA,
            pltpu.SemaphoreType.DMA,
            pltpu.SemaphoreType.DMA,
            pltpu.SemaphoreType.DMA,
        ],
    )
    def k(table_hbm, keys_hbm, out_hbm, keys_v, buf_a, buf_b, tl_a, tl_b,
          sem_ga, sem_gb, sem_ea, sem_eb):
        wid = lax.axis_index("s") * _NC + lax.axis_index("c")
        bbase = wid * _BPW
        pltpu.sync_copy(keys_hbm.at[wid], keys_v)

        def gathers(m, buf, sem):
            return [pltpu.make_async_copy(
                table_hbm.at[keys_v.at[m * _NLC + c]], buf.at[c], sem)
                for c in range(_NLC)]

        def repack(buf, tl):
            for r in range(_CR):
                for l0 in (0, 16, 32, 48, 64, 80, _TAIL - 16):
                    tl[r, pl.ds(l0, 16)] = buf[_NLC - 1, r, pl.ds(l0, 16)]

        def emits(m, buf, tl, sem):
            b0 = bbase + m * _CB
            cps = [pltpu.make_async_copy(
                buf.at[c, pl.ds(q * _T, _T)],
                out_hbm.at[b0 + q, :, pl.ds(c * 128, 128)], sem)
                for c in range(_NLC - 1) for q in range(_CB)]
            cps += [pltpu.make_async_copy(
                tl.at[pl.ds(q * _T, _T)],
                out_hbm.at[b0 + q, :, pl.ds(896, _TAIL)], sem)
                for q in range(_CB)]
            return cps

        for g in gathers(0, buf_a, sem_ga):
            g.start()
        for g in gathers(1, buf_b, sem_gb):
            g.start()

        def step(i, _):
            m = 2 * i
            for g in gathers(m, buf_a, sem_ga):
                g.wait()
            repack(buf_a, tl_a)
            for e in emits(m, buf_a, tl_a, sem_ea):
                e.start()
            for g in gathers(m + 1, buf_b, sem_gb):
                g.wait()
            repack(buf_b, tl_b)
            for e in emits(m + 1, buf_b, tl_b, sem_eb):
                e.start()
            for e in emits(m, buf_a, tl_a, sem_ea):
                e.wait()

            @pl.when(m + 2 < _NCHUNK)
            def _():
                for g in gathers(m + 2, buf_a, sem_ga):
                    g.start()

            for e in emits(m + 1, buf_b, tl_b, sem_eb):
                e.wait()

            @pl.when(m + 3 < _NCHUNK)
            def _():
                for g in gathers(m + 3, buf_b, sem_gb):
                    g.start()

            return _

        lax.fori_loop(0, _NCHUNK // 2, step, None)

    return k(table, keys3)


def kernel(idx, tok_table, pos_table, W, b):
    B, T = idx.shape
    tok_pad = jnp.pad(tok_table, ((0, 0), (0, _CP - _C)))
    pos_pad = jnp.pad(pos_table, ((0, 0), (0, _CP - _C)))
    W_pad = jnp.pad(W, ((0, _CP - _C), (0, _VP - _VOCAB)))
    b_pad = jnp.pad(b, ((0, _VP - _VOCAB),)).reshape(1, _VP)

    table = pl.pallas_call(
        _tt_body,
        grid=(_T, _NLC),
        in_specs=[
            pl.BlockSpec((_VOCAB, _CP), lambda t, c: (0, 0)),
            pl.BlockSpec((_T, _CP), lambda t, c: (0, 0)),
            pl.BlockSpec((_CP, 128), lambda t, c: (0, c)),
            pl.BlockSpec((1, 128), lambda t, c: (0, c)),
        ],
        out_specs=pl.BlockSpec((_VOCAB, 128), lambda t, c: (c * _T + t, 0)),
        out_shape=jax.ShapeDtypeStruct((_NLC * _NKEY, 128), jnp.float32),
    )(tok_pad, pos_pad, W_pad, b_pad)

    kbase = idx + _VOCAB * jnp.arange(_T, dtype=jnp.int32)[None, :]
    keys3 = (kbase.reshape(_NW, _NCHUNK, 1, _CR)
             + _NKEY * jnp.arange(_NLC, dtype=jnp.int32).reshape(1, 1, _NLC, 1)
             ).reshape(_NW, _NCHUNK * _NLC, _CR)

    return _sc_emit(table, keys3)


# speedup vs baseline: 1.0006x; 1.0006x over previous
import functools

import jax
import jax.numpy as jnp
from jax import lax
from jax.experimental import pallas as pl
from jax.experimental.pallas import tpu as pltpu
from jax.experimental.pallas import tpu_sc as plsc

_VOCAB = 1000
_C = 64
_CP = 128
_T = 8
_B = 4096

_NC = 2
_NS = 16
_NW = _NC * _NS
_ROWS = _B * _T
_RPW = _ROWS // _NW
_CHUNK = 128
_NCHUNK = _RPW // _CHUNK

_RB = 128
_SPLIT = 4


def _sc_gather(tok_pad, idx2):
    mesh = plsc.VectorSubcoreMesh(core_axis_name="c", subcore_axis_name="s")

    @functools.partial(
        pl.kernel,
        mesh=mesh,
        out_type=jax.ShapeDtypeStruct((_ROWS, _CP), jnp.float32),
        scratch_types=[
            pltpu.VMEM((_NCHUNK, _CHUNK), jnp.int32),
            pltpu.VMEM((2, _CHUNK, _CP), jnp.float32),
            pltpu.SemaphoreType.DMA,
            pltpu.SemaphoreType.DMA,
        ],
    )
    def k(tok_hbm, idx_hbm, out_hbm, idx_v, buf, sem0, sem1):
        wid = lax.axis_index("s") * _NC + lax.axis_index("c")
        base = wid * _NCHUNK
        pltpu.sync_copy(idx_hbm.at[pl.ds(base, _NCHUNK)], idx_v)
        sems = [sem0, sem1]
        copies = [None, None]
        copies[0] = pltpu.async_copy(
            tok_hbm.at[idx_v.at[0]], buf.at[0], sems[0])
        for j in range(_NCHUNK):
            if j + 1 < _NCHUNK:
                copies[(j + 1) % 2] = pltpu.async_copy(
                    tok_hbm.at[idx_v.at[j + 1]], buf.at[(j + 1) % 2],
                    sems[(j + 1) % 2])
            copies[j % 2].wait()
            pltpu.sync_copy(buf.at[j % 2],
                            out_hbm.at[pl.ds((base + j) * _CHUNK, _CHUNK)])

    return k(tok_pad, idx2)


def _tc_body(x_ref, pos_ref, w_ref, b_ref, o_ref):
    x = x_ref[...].reshape(_RB, _T, _CP) + pos_ref[...][None, :, :]
    y = jnp.dot(x.reshape(_RB * _T, _CP), w_ref[...],
                preferred_element_type=jnp.float32) + b_ref[...]
    o_ref[...] = y


def kernel(idx, tok_table, pos_table, W, b):
    B, T = idx.shape
    tok_pad = jnp.pad(tok_table, ((0, 0), (0, _CP - _C)))
    pos_pad = jnp.pad(pos_table, ((0, 0), (0, _CP - _C)))
    W_pad = jnp.pad(W, ((0, _CP - _C), (0, 0)))
    idx2 = idx.reshape(_NW * _NCHUNK, _CHUNK)
    tok_emb = _sc_gather(tok_pad, idx2)

    b2 = b.reshape(1, _VOCAB)
    rows_k = _ROWS // _SPLIT
    tc = pl.pallas_call(
        _tc_body,
        grid=(rows_k // (_RB * _T),),
        in_specs=[
            pl.BlockSpec((_RB * _T, _CP), lambda i: (i, 0)),
            pl.BlockSpec((_T, _CP), lambda i: (0, 0)),
            pl.BlockSpec((_CP, _VOCAB), lambda i: (0, 0)),
            pl.BlockSpec((1, _VOCAB), lambda i: (0, 0)),
        ],
        out_specs=pl.BlockSpec((_RB * _T, _VOCAB), lambda i: (i, 0)),
        out_shape=jax.ShapeDtypeStruct((rows_k, _VOCAB), jnp.float32),
    )

    parts = []
    for kk in range(_SPLIT):
        chunk = lax.slice_in_dim(tok_emb, kk * rows_k, (kk + 1) * rows_k)
        yk = tc(chunk, pos_pad, W_pad, b2)
        parts.append(yk.reshape(B // _SPLIT, T, _VOCAB))
    return jnp.concatenate(parts, axis=0)


# speedup vs baseline: 1.5267x; 1.5258x over previous
import functools

import jax
import jax.numpy as jnp
from jax import lax
from jax.experimental import pallas as pl
from jax.experimental.pallas import tpu as pltpu
from jax.experimental.pallas import tpu_sc as plsc

_VOCAB = 1000
_C = 64
_CP = 128
_T = 8
_B = 4096

_NC = 2
_NS = 16
_NW = _NC * _NS
_ROWS = _B * _T
_RPW = _ROWS // _NW
_CHUNK = 128
_NCHUNK = _RPW // _CHUNK

_RB = 128


def _sc_gather(tok_pad, idx2):
    mesh = plsc.VectorSubcoreMesh(core_axis_name="c", subcore_axis_name="s")

    @functools.partial(
        pl.kernel,
        mesh=mesh,
        out_type=jax.ShapeDtypeStruct((_ROWS, _CP), jnp.float32),
        scratch_types=[
            pltpu.VMEM((_NCHUNK, _CHUNK), jnp.int32),
            pltpu.VMEM((2, _CHUNK, _CP), jnp.float32),
            pltpu.SemaphoreType.DMA,
            pltpu.SemaphoreType.DMA,
        ],
    )
    def k(tok_hbm, idx_hbm, out_hbm, idx_v, buf, sem0, sem1):
        wid = lax.axis_index("s") * _NC + lax.axis_index("c")
        base = wid * _NCHUNK
        pltpu.sync_copy(idx_hbm.at[pl.ds(base, _NCHUNK)], idx_v)
        sems = [sem0, sem1]
        copies = [None, None]
        copies[0] = pltpu.async_copy(
            tok_hbm.at[idx_v.at[0]], buf.at[0], sems[0])
        for j in range(_NCHUNK):
            if j + 1 < _NCHUNK:
                copies[(j + 1) % 2] = pltpu.async_copy(
                    tok_hbm.at[idx_v.at[j + 1]], buf.at[(j + 1) % 2],
                    sems[(j + 1) % 2])
            copies[j % 2].wait()
            pltpu.sync_copy(buf.at[j % 2],
                            out_hbm.at[pl.ds((base + j) * _CHUNK, _CHUNK)])

    return k(tok_pad, idx2)


def _tc_body(x_ref, pos_ref, w_ref, b_ref, o_ref):
    x = x_ref[...].reshape(_RB, _T, _CP) + pos_ref[...][None, :, :]
    y = jnp.dot(x.reshape(_RB * _T, _CP), w_ref[...],
                preferred_element_type=jnp.float32) + b_ref[...]
    o_ref[...] = y


def kernel(idx, tok_table, pos_table, W, b):
    B, T = idx.shape
    tok_pad = jnp.pad(tok_table, ((0, 0), (0, _CP - _C)))
    pos_pad = jnp.pad(pos_table, ((0, 0), (0, _CP - _C)))
    W_pad = jnp.pad(W, ((0, _CP - _C), (0, 0)))
    idx2 = idx.reshape(_NW * _NCHUNK, _CHUNK)
    tok_emb = _sc_gather(tok_pad, idx2)

    out = pl.pallas_call(
        _tc_body,
        grid=(_ROWS // (_RB * _T),),
        in_specs=[
            pl.BlockSpec((_RB * _T, _CP), lambda i: (i, 0)),
            pl.BlockSpec((_T, _CP), lambda i: (0, 0)),
            pl.BlockSpec((_CP, _VOCAB), lambda i: (0, 0)),
            pl.BlockSpec((1, _VOCAB), lambda i: (0, 0)),
        ],
        out_specs=pl.BlockSpec((_RB * _T, _VOCAB), lambda i: (i, 0)),
        out_shape=jax.ShapeDtypeStruct((_ROWS, _VOCAB), jnp.float32),
    )(tok_emb, pos_pad, W_pad, b.reshape(1, _VOCAB))

    return out.reshape(B, T, _VOCAB)
